# baseline (speedup 1.0000x reference)
import functools
import math

import jax
import jax.numpy as jnp
from jax.experimental import pallas as pl
from jax.experimental.pallas import tpu as pltpu

N = 100000
E = 1600000
C_IN = 8
C_H = 64
NB = 8
INV_SQRT_NN = 1.0 / math.sqrt(16.0)

EDGE_BLK = 4000
NODE_BLK = 2000


def _silu(v):
    return v * jax.nn.sigmoid(v)


def _edge_mlp_body(h_ref,
                   wr10, br10, wr20, br20, wr30,
                   wr11, br11, wr21, br21, wr31,
                   wr12, br12, wr22, br22, wr32,
                   w0_ref, w1_ref, w2_ref):
    h = h_ref[...]

    def chain(wr1, br1, wr2, br2, wr3):
        a = _silu(jnp.dot(h, wr1[...], preferred_element_type=jnp.float32)
                  + br1[...][None, :])
        b = _silu(jnp.dot(a, wr2[...], preferred_element_type=jnp.float32)
                  + br2[...][None, :])
        return jnp.dot(b, wr3[...], preferred_element_type=jnp.float32)

    w0_ref[...] = chain(wr10, br10, wr20, br20, wr30)
    w1_ref[...] = chain(wr11, br11, wr21, br21, wr31)
    w2_ref[...] = chain(wr12, br12, wr22, br22, wr32)


def _edge_mlp(h_edge, packs):
    grid = (E // EDGE_BLK,)
    full = lambda *s: pl.BlockSpec(s, lambda i: (0,) * len(s))
    in_specs = [pl.BlockSpec((EDGE_BLK, NB), lambda i: (i, 0))]
    args = [h_edge]
    for (wr1, br1, wr2, br2, wr3) in packs:
        in_specs += [full(*wr1.shape), full(*br1.shape), full(*wr2.shape),
                     full(*br2.shape), full(*wr3.shape)]
        args += [wr1, br1, wr2, br2, wr3]
    out_specs = [
        pl.BlockSpec((EDGE_BLK, 16), lambda i: (i, 0)),
        pl.BlockSpec((EDGE_BLK, 64), lambda i: (i, 0)),
        pl.BlockSpec((EDGE_BLK, 64), lambda i: (i, 0)),
    ]
    out_shape = [
        jax.ShapeDtypeStruct((E, 16), jnp.float32),
        jax.ShapeDtypeStruct((E, 64), jnp.float32),
        jax.ShapeDtypeStruct((E, 64), jnp.float32),
    ]
    return pl.pallas_call(
        _edge_mlp_body, grid=grid, in_specs=in_specs, out_specs=out_specs,
        out_shape=out_shape)(*args)


def _node_pre_body(x_ref, z_ref, wsc_ref, w1_ref, sc_ref, x1_ref, *, c):
    x = x_ref[...]
    z = z_ref[...]
    acc = jnp.zeros((x.shape[0], 64), jnp.float32)
    for j in range(8):
        acc += jnp.dot(x * z[:, j][:, None], wsc_ref[j],
                       preferred_element_type=jnp.float32)
    sc_ref[...] = acc * (1.0 / math.sqrt(c * 8.0))
    x1_ref[...] = jnp.dot(x, w1_ref[...],
                          preferred_element_type=jnp.float32) * (1.0 / math.sqrt(c))


def _node_pre(x, z, wsc_t, w1):
    c = x.shape[1]
    cx1 = w1.shape[1]
    grid = (N // NODE_BLK,)
    return pl.pallas_call(
        functools.partial(_node_pre_body, c=c),
        grid=grid,
        in_specs=[
            pl.BlockSpec((NODE_BLK, c), lambda i: (i, 0)),
            pl.BlockSpec((NODE_BLK, 8), lambda i: (i, 0)),
            pl.BlockSpec(wsc_t.shape, lambda i: (0, 0, 0)),
            pl.BlockSpec(w1.shape, lambda i: (0, 0)),
        ],
        out_specs=[
            pl.BlockSpec((NODE_BLK, 64), lambda i: (i, 0)),
            pl.BlockSpec((NODE_BLK, cx1), lambda i: (i, 0)),
        ],
        out_shape=[
            jax.ShapeDtypeStruct((N, 64), jnp.float32),
            jax.ShapeDtypeStruct((N, cx1), jnp.float32),
        ])(x, z, wsc_t, w1)


def _combine_body(sc_ref, agg_ref, w2_ref, out_ref, *, scale):
    y = jnp.dot(agg_ref[...], w2_ref[...], preferred_element_type=jnp.float32)
    out_ref[...] = _silu(sc_ref[...] + y * scale)


def _combine(sc, agg, w2, c):
    scale = (1.0 / math.sqrt(c)) * INV_SQRT_NN
    cagg = agg.shape[1]
    grid = (N // NODE_BLK,)
    return pl.pallas_call(
        functools.partial(_combine_body, scale=scale),
        grid=grid,
        in_specs=[
            pl.BlockSpec((NODE_BLK, 64), lambda i: (i, 0)),
            pl.BlockSpec((NODE_BLK, cagg), lambda i: (i, 0)),
            pl.BlockSpec(w2.shape, lambda i: (0, 0)),
        ],
        out_specs=pl.BlockSpec((NODE_BLK, 64), lambda i: (i, 0)),
        out_shape=jax.ShapeDtypeStruct((N, 64), jnp.float32))(sc, agg, w2)


def _final_body(x_ref, z_ref, wz_ref, out_ref):
    t = jnp.dot(z_ref[...], wz_ref[...], preferred_element_type=jnp.float32)
    out_ref[...] = jnp.sum(x_ref[...] * t, axis=1, keepdims=True) * (
        1.0 / math.sqrt(64.0 * 8.0))


def _final(x, z, wz):
    grid = (N // NODE_BLK,)
    return pl.pallas_call(
        _final_body, grid=grid,
        in_specs=[
            pl.BlockSpec((NODE_BLK, 64), lambda i: (i, 0)),
            pl.BlockSpec((NODE_BLK, 8), lambda i: (i, 0)),
            pl.BlockSpec((8, 64), lambda i: (0, 0)),
        ],
        out_specs=pl.BlockSpec((NODE_BLK, 1), lambda i: (i, 0)),
        out_shape=jax.ShapeDtypeStruct((N, 1), jnp.float32))(x, z, wz)


def kernel(h_node_x, h_node_z, edge_index, edge_attr, h_edge,
           Wsc0, W10, Wr10, br10, Wr20, br20, Wr30, W20,
           Wsc1, W11, Wr11, br11, Wr21, br21, Wr31, W21,
           Wsc2, W12, Wr12, br12, Wr22, br22, Wr32, W22,
           Wout):
    src = edge_index[0]
    dst = edge_index[1]

    W10p = jnp.pad(W10, ((0, 0), (0, 8)))
    Wr30p = jnp.pad(Wr30, ((0, 0), (0, 8)))
    W20p = jnp.pad(W20, ((0, 8), (0, 0)))

    w0, w1, w2 = _edge_mlp(h_edge, [
        (Wr10, br10, Wr20, br20, Wr30p),
        (Wr11, br11, Wr21, br21, Wr31),
        (Wr12, br12, Wr22, br22, Wr32),
    ])

    x = h_node_x
    z = h_node_z
    layer_data = [
        (Wsc0, W10p, w0, W20p, 8),
        (Wsc1, W11, w1, W21, 64),
        (Wsc2, W12, w2, W22, 64),
    ]
    for (Wsc, W1, w, W2, c) in layer_data:
        wsc_t = jnp.transpose(Wsc, (1, 0, 2))
        sc, x1 = _node_pre(x, z, wsc_t, W1)
        m = x1[src] * w
        agg = jnp.zeros((N, x1.shape[1]), jnp.float32).at[dst].add(m)
        x = _combine(sc, agg, W2, c)

    wz = jnp.transpose(Wout[:, :, 0])
    return _final(x, z, wz)


# speedup vs baseline: 2.2664x; 2.2664x over previous
import functools
import math

import jax
import jax.numpy as jnp
from jax import lax
from jax.experimental import pallas as pl
from jax.experimental.pallas import tpu as pltpu
from jax.experimental.pallas import tpu_sc as plsc

N = 100000
E = 1600000
C_IN = 8
C_H = 64
NB = 8
INV_SQRT_NN = 1.0 / math.sqrt(16.0)

EDGE_BLK = 4000
NODE_BLK = 2000


def _silu(v):
    return v * jax.nn.sigmoid(v)


def _edge_mlp_body(h_ref,
                   wr10, br10, wr20, br20, wr30,
                   wr11, br11, wr21, br21, wr31,
                   wr12, br12, wr22, br22, wr32,
                   w0_ref, *wc_refs):
    h = h_ref[...]

    def chain(wr1, br1, wr2, br2, wr3):
        a = _silu(jnp.dot(h, wr1[...], preferred_element_type=jnp.float32)
                  + br1[...][None, :])
        b = _silu(jnp.dot(a, wr2[...], preferred_element_type=jnp.float32)
                  + br2[...][None, :])
        return jnp.dot(b, wr3[...], preferred_element_type=jnp.float32)

    w0_ref[...] = chain(wr10, br10, wr20, br20, wr30)
    w1 = chain(wr11, br11, wr21, br21, wr31)
    w2 = chain(wr12, br12, wr22, br22, wr32)
    for k in range(4):
        wc_refs[k][...] = w1[:, 16 * k:16 * (k + 1)]
        wc_refs[4 + k][...] = w2[:, 16 * k:16 * (k + 1)]


def _edge_mlp(h_edge, packs):
    grid = (E // EDGE_BLK,)
    full = lambda *s: pl.BlockSpec(s, lambda i: (0,) * len(s))
    in_specs = [pl.BlockSpec((EDGE_BLK, NB), lambda i: (i, 0))]
    args = [h_edge]
    for (wr1, br1, wr2, br2, wr3) in packs:
        in_specs += [full(*wr1.shape), full(*br1.shape), full(*wr2.shape),
                     full(*br2.shape), full(*wr3.shape)]
        args += [wr1, br1, wr2, br2, wr3]
    out_specs = [pl.BlockSpec((EDGE_BLK, 16), lambda i: (i, 0))] * 9
    out_shape = [jax.ShapeDtypeStruct((E, 16), jnp.float32)] * 9
    return pl.pallas_call(
        _edge_mlp_body, grid=grid, in_specs=in_specs, out_specs=out_specs,
        out_shape=out_shape)(*args)


def _node_pre_body(x_ref, z_ref, wsc_ref, w1_ref, sc_ref, x1_ref, *, c):
    x = x_ref[...]
    z = z_ref[...]
    acc = jnp.zeros((x.shape[0], 64), jnp.float32)
    for j in range(8):
        acc += jnp.dot(x * z[:, j][:, None], wsc_ref[j],
                       preferred_element_type=jnp.float32)
    sc_ref[...] = acc * (1.0 / math.sqrt(c * 8.0))
    x1_ref[...] = jnp.dot(x, w1_ref[...],
                          preferred_element_type=jnp.float32) * (1.0 / math.sqrt(c))


def _node_pre(x, z, wsc_t, w1):
    c = x.shape[1]
    cx1 = w1.shape[1]
    grid = (N // NODE_BLK,)
    return pl.pallas_call(
        functools.partial(_node_pre_body, c=c),
        grid=grid,
        in_specs=[
            pl.BlockSpec((NODE_BLK, c), lambda i: (i, 0)),
            pl.BlockSpec((NODE_BLK, 8), lambda i: (i, 0)),
            pl.BlockSpec(wsc_t.shape, lambda i: (0, 0, 0)),
            pl.BlockSpec(w1.shape, lambda i: (0, 0)),
        ],
        out_specs=[
            pl.BlockSpec((NODE_BLK, 64), lambda i: (i, 0)),
            pl.BlockSpec((NODE_BLK, cx1), lambda i: (i, 0)),
        ],
        out_shape=[
            jax.ShapeDtypeStruct((N, 64), jnp.float32),
            jax.ShapeDtypeStruct((N, cx1), jnp.float32),
        ])(x, z, wsc_t, w1)


def _combine_body(*refs, scale, npairs):
    sc_ref = refs[0]
    out_ref = refs[1 + 2 * npairs]
    y = jnp.zeros((NODE_BLK, 64), jnp.float32)
    for p in range(npairs):
        agg_ref = refs[1 + 2 * p]
        w2_ref = refs[2 + 2 * p]
        y += jnp.dot(agg_ref[...], w2_ref[...], preferred_element_type=jnp.float32)
    out_ref[...] = _silu(sc_ref[...] + y * scale)


def _combine(sc, pairs, c):
    scale = (1.0 / math.sqrt(c)) * INV_SQRT_NN
    grid = (N // NODE_BLK,)
    in_specs = [pl.BlockSpec((NODE_BLK, 64), lambda i: (i, 0))]
    args = [sc]
    for (agg, w2) in pairs:
        in_specs += [
            pl.BlockSpec((NODE_BLK, agg.shape[1]), lambda i: (i, 0)),
            pl.BlockSpec(w2.shape, lambda i: (0, 0)),
        ]
        args += [agg, w2]
    return pl.pallas_call(
        functools.partial(_combine_body, scale=scale, npairs=len(pairs)),
        grid=grid,
        in_specs=in_specs,
        out_specs=pl.BlockSpec((NODE_BLK, 64), lambda i: (i, 0)),
        out_shape=jax.ShapeDtypeStruct((N, 64), jnp.float32))(*args)


GRP = 128
GPB = 4
BLK = GRP * GPB
NBLK = E // BLK
ROWS_MAIN = 6256
ROWS_LAST = N - 15 * ROWS_MAIN


def _sc_process_block(x1_ref, w_ref, blk,
                      src2d, dst2d, accum, sidx, didx, xg, wv, msg, sem, sem2):
    r0 = blk * GPB
    e0 = blk * BLK
    pltpu.sync_copy(src2d.at[pl.ds(r0, GPB), :], sidx)
    pltpu.sync_copy(dst2d.at[pl.ds(r0, GPB), :], didx)
    wcopy = pltpu.async_copy(w_ref.at[pl.ds(e0, BLK), :], wv, sem2)
    gathers = [
        pltpu.async_copy(x1_ref.at[sidx.at[g]],
                         xg.at[pl.ds(g * GRP, GRP), :], sem)
        for g in range(GPB)
    ]
    for cp in gathers:
        cp.wait()
    wcopy.wait()

    def mul_body(r, carry):
        msg[r, :] = xg[r, :] * wv[r, :]
        return carry

    lax.fori_loop(0, BLK, mul_body, 0)
    for g in range(GPB):
        pltpu.sync_copy(msg.at[pl.ds(g * GRP, GRP), :],
                        accum.at[didx.at[g]], add=True)


def _sc_rowcopy(src_ref, dst_ref, sid):
    @pl.when(sid < 15)
    def _():
        n0 = pl.multiple_of(sid * ROWS_MAIN, 16)
        pltpu.sync_copy(src_ref.at[pl.ds(n0, ROWS_MAIN), :],
                        dst_ref.at[pl.ds(n0, ROWS_MAIN), :])

    @pl.when(sid == 15)
    def _():
        pltpu.sync_copy(src_ref.at[pl.ds(15 * ROWS_MAIN, ROWS_LAST), :],
                        dst_ref.at[pl.ds(15 * ROWS_MAIN, ROWS_LAST), :])


_SC_SCRATCH = [
    pltpu.VMEM((GPB, GRP), jnp.int32),
    pltpu.VMEM((GPB, GRP), jnp.int32),
    pltpu.VMEM((BLK, 16), jnp.float32),
    pltpu.VMEM((BLK, 16), jnp.float32),
    pltpu.VMEM((BLK, 16), jnp.float32),
    pltpu.VMEM_SHARED((N, 16), jnp.float32),
    pltpu.SemaphoreType.DMA,
    pltpu.SemaphoreType.DMA,
]


def _sc_edge_stage_wide(x1c, src2d, dst2d, wc, zeros):
    mesh = plsc.VectorSubcoreMesh(core_axis_name="c", subcore_axis_name="s")

    @functools.partial(
        pl.kernel,
        out_type=[jax.ShapeDtypeStruct((N, 16), jnp.float32)] * 4,
        mesh=mesh,
        scratch_types=list(_SC_SCRATCH),
        compiler_params=pltpu.CompilerParams(use_tc_tiling_on_sc=False),
    )
    def body(x10, x11, x12, x13, src_r, dst_r, w0_r, w1_r, w2_r, w3_r,
             zeros_r, agg0, agg1, agg2, agg3,
             sidx, didx, xg, wv, msg, accum, sem, sem2):
        cid = lax.axis_index("c")
        sid = lax.axis_index("s")

        def do_chunk(x1_ref, w_ref, agg_ref):
            _sc_rowcopy(zeros_r, accum, sid)
            plsc.subcore_barrier()

            def blk_body(i, carry):
                blk = i * 16 + sid

                @pl.when(blk < NBLK)
                def _():
                    _sc_process_block(x1_ref, w_ref, blk, src_r, dst_r,
                                      accum, sidx, didx, xg, wv, msg, sem,
                                      sem2)
                return carry

            lax.fori_loop(0, (NBLK + 15) // 16, blk_body, 0)
            plsc.subcore_barrier()
            _sc_rowcopy(accum, agg_ref, sid)

        @pl.when(cid == 0)
        def _():
            do_chunk(x10, w0_r, agg0)
            do_chunk(x11, w1_r, agg1)

        @pl.when(cid == 1)
        def _():
            do_chunk(x12, w2_r, agg2)
            do_chunk(x13, w3_r, agg3)

    return body(x1c[0], x1c[1], x1c[2], x1c[3], src2d, dst2d,
                wc[0], wc[1], wc[2], wc[3], zeros)


def _sc_edge_stage_narrow(x1p, src2d, dst2d, w, zeros):
    mesh = plsc.VectorSubcoreMesh(core_axis_name="c", subcore_axis_name="s")

    @functools.partial(
        pl.kernel,
        out_type=[jax.ShapeDtypeStruct((N, 16), jnp.float32)] * 2,
        mesh=mesh,
        scratch_types=list(_SC_SCRATCH),
        compiler_params=pltpu.CompilerParams(use_tc_tiling_on_sc=False),
    )
    def body(x1_r, src_r, dst_r, w_r, zeros_r, agg_a, agg_b,
             sidx, didx, xg, wv, msg, accum, sem, sem2):
        cid = lax.axis_index("c")
        sid = lax.axis_index("s")
        wid = cid * 16 + sid

        _sc_rowcopy(zeros_r, accum, sid)
        plsc.subcore_barrier()

        def blk_body(i, carry):
            blk = i * 32 + wid

            @pl.when(blk < NBLK)
            def _():
                _sc_process_block(x1_r, w_r, blk, src_r, dst_r, accum,
                                  sidx, didx, xg, wv, msg, sem, sem2)
            return carry

        lax.fori_loop(0, (NBLK + 31) // 32, blk_body, 0)
        plsc.subcore_barrier()

        @pl.when(cid == 0)
        def _():
            _sc_rowcopy(accum, agg_a, sid)

        @pl.when(cid == 1)
        def _():
            _sc_rowcopy(accum, agg_b, sid)

    return body(x1p, src2d, dst2d, w, zeros)


def _final_body(x_ref, z_ref, wz_ref, out_ref):
    t = jnp.dot(z_ref[...], wz_ref[...], preferred_element_type=jnp.float32)
    out_ref[...] = jnp.sum(x_ref[...] * t, axis=1, keepdims=True) * (
        1.0 / math.sqrt(64.0 * 8.0))


def _final(x, z, wz):
    grid = (N // NODE_BLK,)
    return pl.pallas_call(
        _final_body, grid=grid,
        in_specs=[
            pl.BlockSpec((NODE_BLK, 64), lambda i: (i, 0)),
            pl.BlockSpec((NODE_BLK, 8), lambda i: (i, 0)),
            pl.BlockSpec((8, 64), lambda i: (0, 0)),
        ],
        out_specs=pl.BlockSpec((NODE_BLK, 1), lambda i: (i, 0)),
        out_shape=jax.ShapeDtypeStruct((N, 1), jnp.float32))(x, z, wz)


def kernel(h_node_x, h_node_z, edge_index, edge_attr, h_edge,
           Wsc0, W10, Wr10, br10, Wr20, br20, Wr30, W20,
           Wsc1, W11, Wr11, br11, Wr21, br21, Wr31, W21,
           Wsc2, W12, Wr12, br12, Wr22, br22, Wr32, W22,
           Wout):
    src = edge_index[0]
    dst = edge_index[1]

    W10p = jnp.pad(W10, ((0, 0), (0, 8)))
    Wr30p = jnp.pad(Wr30, ((0, 0), (0, 8)))
    W20p = jnp.pad(W20, ((0, 8), (0, 0)))

    w_outs = _edge_mlp(h_edge, [
        (Wr10, br10, Wr20, br20, Wr30p),
        (Wr11, br11, Wr21, br21, Wr31),
        (Wr12, br12, Wr22, br22, Wr32),
    ])
    w0 = w_outs[0]
    w1c = w_outs[1:5]
    w2c = w_outs[5:9]

    src2d = src.reshape(E // GRP, GRP)
    dst2d = dst.reshape(E // GRP, GRP)
    zeros = jnp.zeros((N, 16), jnp.float32)

    x = h_node_x
    z = h_node_z

    sc, x1 = _node_pre(x, z, jnp.transpose(Wsc0, (1, 0, 2)), W10p)
    agg_a, agg_b = _sc_edge_stage_narrow(x1, src2d, dst2d, w0, zeros)
    x = _combine(sc, [(agg_a, W20p), (agg_b, W20p)], 8)

    for (Wsc, W1, wc, W2) in ((Wsc1, W11, w1c, W21), (Wsc2, W12, w2c, W22)):
        sc, x1 = _node_pre(x, z, jnp.transpose(Wsc, (1, 0, 2)), W1)
        x1c = [jnp.asarray(x1[:, 16 * k:16 * (k + 1)]) for k in range(4)]
        aggs = _sc_edge_stage_wide(x1c, src2d, dst2d, wc, zeros)
        x = _combine(sc, [(aggs[k], W2[16 * k:16 * (k + 1), :])
                          for k in range(4)], 64)

    wz = jnp.transpose(Wout[:, :, 0])
    return _final(x, z, wz)


# speedup vs baseline: 3.4647x; 1.5287x over previous
import functools
import math

import jax
import jax.numpy as jnp
from jax import lax
from jax.experimental import pallas as pl
from jax.experimental.pallas import tpu as pltpu
from jax.experimental.pallas import tpu_sc as plsc

N = 100000
E = 1600000
NB = 8
INV_SQRT_NN = 1.0 / math.sqrt(16.0)

EDGE_BLK = 8000
NODE_BLK = 2000

BF = jnp.bfloat16


def _silu(v):
    return v * jax.nn.sigmoid(v)


def _edge_mlp_body(h_ref,
                   wr10, br10, wr20, br20, wr30,
                   wr11, br11, wr21, br21, wr31,
                   wr12, br12, wr22, br22, wr32,
                   w0_ref, w1a_ref, w1b_ref, w2a_ref, w2b_ref):
    h = h_ref[...]

    def chain(wr1, br1, wr2, br2, wr3):
        a = _silu(jnp.dot(h, wr1[...], preferred_element_type=jnp.float32)
                  + br1[...][None, :])
        b = _silu(jnp.dot(a, wr2[...], preferred_element_type=jnp.float32)
                  + br2[...][None, :])
        return jnp.dot(b.astype(BF), wr3[...].astype(BF),
                       preferred_element_type=jnp.float32)

    w0_ref[...] = chain(wr10, br10, wr20, br20, wr30).astype(BF)
    w1 = chain(wr11, br11, wr21, br21, wr31)
    w1a_ref[...] = w1[:, :32].astype(BF)
    w1b_ref[...] = w1[:, 32:].astype(BF)
    w2 = chain(wr12, br12, wr22, br22, wr32)
    w2a_ref[...] = w2[:, :32].astype(BF)
    w2b_ref[...] = w2[:, 32:].astype(BF)


def _edge_mlp(h_edge, packs):
    grid = (E // EDGE_BLK,)
    full = lambda *s: pl.BlockSpec(s, lambda i: (0,) * len(s))
    in_specs = [pl.BlockSpec((EDGE_BLK, NB), lambda i: (i, 0))]
    args = [h_edge]
    for (wr1, br1, wr2, br2, wr3) in packs:
        in_specs += [full(*wr1.shape), full(*br1.shape), full(*wr2.shape),
                     full(*br2.shape), full(*wr3.shape)]
        args += [wr1, br1, wr2, br2, wr3]
    out_specs = [pl.BlockSpec((EDGE_BLK, 32), lambda i: (i, 0))] * 5
    out_shape = [jax.ShapeDtypeStruct((E, 32), BF)] * 5
    return pl.pallas_call(
        _edge_mlp_body, grid=grid, in_specs=in_specs, out_specs=out_specs,
        out_shape=out_shape)(*args)


def _node_pre_body(x_ref, z_ref, wsc_ref, w1_ref, sc_ref, *x1_refs, c):
    x = x_ref[...]
    z = z_ref[...]
    acc = jnp.zeros((x.shape[0], 64), jnp.float32)
    for j in range(8):
        acc += jnp.dot(x * z[:, j][:, None], wsc_ref[j],
                       preferred_element_type=jnp.float32)
    sc_ref[...] = acc * (1.0 / math.sqrt(c * 8.0))
    x1 = jnp.dot(x, w1_ref[...],
                 preferred_element_type=jnp.float32) * (1.0 / math.sqrt(c))
    for k, ref in enumerate(x1_refs):
        ref[...] = x1[:, 32 * k:32 * (k + 1)].astype(BF)


def _node_pre(x, z, wsc_t, w1):
    c = x.shape[1]
    nchunk = w1.shape[1] // 32
    grid = (N // NODE_BLK,)
    return pl.pallas_call(
        functools.partial(_node_pre_body, c=c),
        grid=grid,
        in_specs=[
            pl.BlockSpec((NODE_BLK, c), lambda i: (i, 0)),
            pl.BlockSpec((NODE_BLK, 8), lambda i: (i, 0)),
            pl.BlockSpec(wsc_t.shape, lambda i: (0, 0, 0)),
            pl.BlockSpec(w1.shape, lambda i: (0, 0)),
        ],
        out_specs=[pl.BlockSpec((NODE_BLK, 64), lambda i: (i, 0))] +
                  [pl.BlockSpec((NODE_BLK, 32), lambda i: (i, 0))] * nchunk,
        out_shape=[jax.ShapeDtypeStruct((N, 64), jnp.float32)] +
                  [jax.ShapeDtypeStruct((N, 32), BF)] * nchunk,
    )(x, z, wsc_t, w1)


def _combine_body(*refs, scale, npairs):
    sc_ref = refs[0]
    out_ref = refs[1 + 2 * npairs]
    y = jnp.zeros((NODE_BLK, 64), jnp.float32)
    for p in range(npairs):
        agg = refs[1 + 2 * p][...]
        w2 = refs[2 + 2 * p][...]
        y += jnp.dot(agg, w2.astype(BF), preferred_element_type=jnp.float32)
    out_ref[...] = _silu(sc_ref[...] + y * scale)


def _combine(sc, pairs, c):
    scale = (1.0 / math.sqrt(c)) * INV_SQRT_NN
    grid = (N // NODE_BLK,)
    in_specs = [pl.BlockSpec((NODE_BLK, 64), lambda i: (i, 0))]
    args = [sc]
    for (agg, w2) in pairs:
        in_specs += [
            pl.BlockSpec((NODE_BLK, 32), lambda i: (i, 0)),
            pl.BlockSpec(w2.shape, lambda i: (0, 0)),
        ]
        args += [agg, w2]
    return pl.pallas_call(
        functools.partial(_combine_body, scale=scale, npairs=len(pairs)),
        grid=grid,
        in_specs=in_specs,
        out_specs=pl.BlockSpec((NODE_BLK, 64), lambda i: (i, 0)),
        out_shape=jax.ShapeDtypeStruct((N, 64), jnp.float32))(*args)


def _final_body(x_ref, z_ref, wz_ref, out_ref):
    t = jnp.dot(z_ref[...], wz_ref[...], preferred_element_type=jnp.float32)
    out_ref[...] = jnp.sum(x_ref[...] * t, axis=1, keepdims=True) * (
        1.0 / math.sqrt(64.0 * 8.0))


def _final(x, z, wz):
    grid = (N // NODE_BLK,)
    return pl.pallas_call(
        _final_body, grid=grid,
        in_specs=[
            pl.BlockSpec((NODE_BLK, 64), lambda i: (i, 0)),
            pl.BlockSpec((NODE_BLK, 8), lambda i: (i, 0)),
            pl.BlockSpec((8, 64), lambda i: (0, 0)),
        ],
        out_specs=pl.BlockSpec((NODE_BLK, 1), lambda i: (i, 0)),
        out_shape=jax.ShapeDtypeStruct((N, 1), jnp.float32))(x, z, wz)


GRP = 128
GPB = 4
BLK = GRP * GPB
NBLK = E // BLK
ROWS_MAIN = 6256
ROWS_LAST = N - 15 * ROWS_MAIN

_SC_SCRATCH = [
    pltpu.VMEM((GPB, GRP), jnp.int32),
    pltpu.VMEM((GPB, GRP), jnp.int32),
    pltpu.VMEM((BLK, 32), BF),
    pltpu.VMEM((BLK, 32), BF),
    pltpu.VMEM((BLK, 32), BF),
    pltpu.VMEM_SHARED((N, 32), BF),
    pltpu.SemaphoreType.DMA,
    pltpu.SemaphoreType.DMA,
]


def _sc_process_block(x1_ref, w_ref, blk,
                      src2d, dst2d, accum, sidx, didx, xg, wv, msg, sem, sem2):
    r0 = blk * GPB
    e0 = blk * BLK
    pltpu.sync_copy(src2d.at[pl.ds(r0, GPB), :], sidx)
    pltpu.sync_copy(dst2d.at[pl.ds(r0, GPB), :], didx)
    wcopy = pltpu.async_copy(w_ref.at[pl.ds(e0, BLK), :], wv, sem2)
    gathers = [
        pltpu.async_copy(x1_ref.at[sidx.at[g]],
                         xg.at[pl.ds(g * GRP, GRP), :], sem)
        for g in range(GPB)
    ]
    for cp in gathers:
        cp.wait()
    wcopy.wait()

    def mul_body(r, carry):
        msg[r, :] = xg[r, :] * wv[r, :]
        return carry

    lax.fori_loop(0, BLK, mul_body, 0)
    for g in range(GPB):
        pltpu.sync_copy(msg.at[pl.ds(g * GRP, GRP), :],
                        accum.at[didx.at[g]], add=True)


def _sc_rowcopy(src_ref, dst_ref, sid):
    @pl.when(sid < 15)
    def _():
        n0 = pl.multiple_of(sid * ROWS_MAIN, 16)
        pltpu.sync_copy(src_ref.at[pl.ds(n0, ROWS_MAIN), :],
                        dst_ref.at[pl.ds(n0, ROWS_MAIN), :])

    @pl.when(sid == 15)
    def _():
        pltpu.sync_copy(src_ref.at[pl.ds(15 * ROWS_MAIN, ROWS_LAST), :],
                        dst_ref.at[pl.ds(15 * ROWS_MAIN, ROWS_LAST), :])


def _sc_edge_stage_wide(x1a, x1b, src2d, dst2d, wa, wb, zeros):
    mesh = plsc.VectorSubcoreMesh(core_axis_name="c", subcore_axis_name="s")

    @functools.partial(
        pl.kernel,
        out_type=[jax.ShapeDtypeStruct((N, 32), BF)] * 2,
        mesh=mesh,
        scratch_types=list(_SC_SCRATCH),
        compiler_params=pltpu.CompilerParams(use_tc_tiling_on_sc=False),
    )
    def body(x1a_r, x1b_r, src_r, dst_r, wa_r, wb_r, zeros_r,
             agg_a, agg_b,
             sidx, didx, xg, wv, msg, accum, sem, sem2):
        cid = lax.axis_index("c")
        sid = lax.axis_index("s")

        def do_chunk(x1_ref, w_ref, agg_ref):
            _sc_rowcopy(zeros_r, accum, sid)
            plsc.subcore_barrier()

            def blk_body(i, carry):
                blk = i * 16 + sid

                @pl.when(blk < NBLK)
                def _():
                    _sc_process_block(x1_ref, w_ref, blk, src_r, dst_r,
                                      accum, sidx, didx, xg, wv, msg, sem,
                                      sem2)
                return carry

            lax.fori_loop(0, (NBLK + 15) // 16, blk_body, 0)
            plsc.subcore_barrier()
            _sc_rowcopy(accum, agg_ref, sid)

        @pl.when(cid == 0)
        def _():
            do_chunk(x1a_r, wa_r, agg_a)

        @pl.when(cid == 1)
        def _():
            do_chunk(x1b_r, wb_r, agg_b)

    return body(x1a, x1b, src2d, dst2d, wa, wb, zeros)


def _sc_edge_stage_narrow(x1p, src2d, dst2d, w, zeros):
    mesh = plsc.VectorSubcoreMesh(core_axis_name="c", subcore_axis_name="s")

    @functools.partial(
        pl.kernel,
        out_type=[jax.ShapeDtypeStruct((N, 32), BF)] * 2,
        mesh=mesh,
        scratch_types=list(_SC_SCRATCH),
        compiler_params=pltpu.CompilerParams(use_tc_tiling_on_sc=False),
    )
    def body(x1_r, src_r, dst_r, w_r, zeros_r, agg_a, agg_b,
             sidx, didx, xg, wv, msg, accum, sem, sem2):
        cid = lax.axis_index("c")
        sid = lax.axis_index("s")
        wid = cid * 16 + sid

        _sc_rowcopy(zeros_r, accum, sid)
        plsc.subcore_barrier()

        def blk_body(i, carry):
            blk = i * 32 + wid

            @pl.when(blk < NBLK)
            def _():
                _sc_process_block(x1_r, w_r, blk, src_r, dst_r, accum,
                                  sidx, didx, xg, wv, msg, sem, sem2)
            return carry

        lax.fori_loop(0, (NBLK + 31) // 32, blk_body, 0)
        plsc.subcore_barrier()

        @pl.when(cid == 0)
        def _():
            _sc_rowcopy(accum, agg_a, sid)

        @pl.when(cid == 1)
        def _():
            _sc_rowcopy(accum, agg_b, sid)

    return body(x1p, src2d, dst2d, w, zeros)


def kernel(h_node_x, h_node_z, edge_index, edge_attr, h_edge,
           Wsc0, W10, Wr10, br10, Wr20, br20, Wr30, W20,
           Wsc1, W11, Wr11, br11, Wr21, br21, Wr31, W21,
           Wsc2, W12, Wr12, br12, Wr22, br22, Wr32, W22,
           Wout):
    src = edge_index[0]
    dst = edge_index[1]

    W10p = jnp.pad(W10, ((0, 0), (0, 24)))
    Wr30p = jnp.pad(Wr30, ((0, 0), (0, 24)))
    W20p = jnp.pad(W20, ((0, 24), (0, 0)))

    w0, w1a, w1b, w2a, w2b = _edge_mlp(h_edge, [
        (Wr10, br10, Wr20, br20, Wr30p),
        (Wr11, br11, Wr21, br21, Wr31),
        (Wr12, br12, Wr22, br22, Wr32),
    ])

    src2d = src.reshape(E // GRP, GRP)
    dst2d = dst.reshape(E // GRP, GRP)
    zeros = jnp.zeros((N, 32), BF)

    x = h_node_x
    z = h_node_z

    sc, x1p = _node_pre(x, z, jnp.transpose(Wsc0, (1, 0, 2)), W10p)
    agg_a, agg_b = _sc_edge_stage_narrow(x1p, src2d, dst2d, w0, zeros)
    x = _combine(sc, [(agg_a, W20p), (agg_b, W20p)], 8)

    for (Wsc, W1, wa, wb, W2) in ((Wsc1, W11, w1a, w1b, W21),
                                  (Wsc2, W12, w2a, w2b, W22)):
        sc, x1a, x1b = _node_pre(x, z, jnp.transpose(Wsc, (1, 0, 2)), W1)
        agg_a, agg_b = _sc_edge_stage_wide(x1a, x1b, src2d, dst2d, wa, wb,
                                           zeros)
        x = _combine(sc, [(agg_a, W2[:32, :]), (agg_b, W2[32:, :])], 64)

    wz = jnp.transpose(Wout[:, :, 0])
    return _final(x, z, wz)
